# baseline (speedup 1.0000x reference)
import functools

import jax
import jax.numpy as jnp
from jax import lax
from jax.experimental import pallas as pl
from jax.experimental.pallas import tpu as pltpu
from jax.experimental.pallas import tpu_sc as plsc

B = 16384
N_CAT = 26
N_CONT = 13
D = 64

NC = 2
NS = 16
NW = NC * NS
ROWS_W = B * N_CAT // NW
NB = 16
CHUNK = NB * N_CAT
NCHUNK = ROWS_W // CHUNK


def _cat_sc_kernel(idx_hbm, table_hbm, bias_hbm, out_hbm,
                   idx0, idx1, rows0, rows1, bias_v, s0, s1):
    wid = lax.axis_index("s") * NC + lax.axis_index("c")
    base = wid * ROWS_W
    pltpu.sync_copy(bias_hbm, bias_v)

    def fire(c, idx_v, rows_v, sem):
        off = base + c * CHUNK
        pltpu.sync_copy(idx_hbm.at[pl.ds(off, CHUNK)], idx_v)
        pltpu.make_async_copy(table_hbm.at[idx_v], rows_v, sem).start()

    def wait(idx_v, rows_v, sem):
        pltpu.make_async_copy(table_hbm.at[idx_v], rows_v, sem).wait()

    def add_bias(rows_v):
        def body(t, carry):
            r0 = t * N_CAT
            for j in range(N_CAT):
                for v in range(D // 16):
                    sl = pl.ds(v * 16, 16)
                    rows_v[r0 + j, sl] = rows_v[r0 + j, sl] + bias_v[j, sl]
            return carry
        lax.fori_loop(0, NB, body, 0)

    def write(c, rows_v):
        off = base + c * CHUNK
        pltpu.sync_copy(rows_v, out_hbm.at[pl.ds(off, CHUNK)])

    fire(0, idx0, rows0, s0)

    def body(i, carry):
        c0 = 2 * i
        fire(c0 + 1, idx1, rows1, s1)
        wait(idx0, rows0, s0)
        add_bias(rows0)
        write(c0, rows0)

        @pl.when(i < NCHUNK // 2 - 1)
        def _():
            fire(c0 + 2, idx0, rows0, s0)

        wait(idx1, rows1, s1)
        add_bias(rows1)
        write(c0 + 1, rows1)
        return carry

    lax.fori_loop(0, NCHUNK // 2, body, 0)


@jax.jit
def _cat_call(idx_flat, table, cat_bias):
    mesh = plsc.VectorSubcoreMesh(core_axis_name="c", subcore_axis_name="s",
                                  num_cores=NC, num_subcores=NS)
    return pl.kernel(
        _cat_sc_kernel,
        out_type=jax.ShapeDtypeStruct((B * N_CAT, D), jnp.float32),
        mesh=mesh,
        scratch_types=[
            pltpu.VMEM((CHUNK,), jnp.int32),
            pltpu.VMEM((CHUNK,), jnp.int32),
            pltpu.VMEM((CHUNK, D), jnp.float32),
            pltpu.VMEM((CHUNK, D), jnp.float32),
            pltpu.VMEM((N_CAT, D), jnp.float32),
            pltpu.SemaphoreType.DMA,
            pltpu.SemaphoreType.DMA,
        ],
        compiler_params=pltpu.CompilerParams(use_tc_tiling_on_sc=False),
    )(idx_flat, table, cat_bias)


BLK = 2048


def _cont_tc_kernel(xc_ref, g_ref, b_ref, w_ref, cb_ref, o_ref):
    xc = xc_ref[...]
    mu = jnp.mean(xc, axis=1, keepdims=True)
    var = jnp.mean((xc - mu) ** 2, axis=1, keepdims=True)
    xcn = (xc - mu) * lax.rsqrt(var + 1e-5)
    xcn = xcn * g_ref[...] + b_ref[...]
    for j in range(N_CONT):
        o_ref[:, j, :] = (xcn[:, j:j + 1] * w_ref[j:j + 1, :]
                          + cb_ref[j:j + 1, :])


@jax.jit
def _cont_call(xc, ln_gamma, ln_beta, cont_w, cont_b):
    grid = (B // BLK,)
    return pl.pallas_call(
        _cont_tc_kernel,
        grid=grid,
        in_specs=[
            pl.BlockSpec((BLK, N_CONT), lambda i: (i, 0)),
            pl.BlockSpec((1, N_CONT), lambda i: (0, 0)),
            pl.BlockSpec((1, N_CONT), lambda i: (0, 0)),
            pl.BlockSpec((N_CONT, D), lambda i: (0, 0)),
            pl.BlockSpec((N_CONT, D), lambda i: (0, 0)),
        ],
        out_specs=pl.BlockSpec((BLK, N_CONT, D), lambda i: (i, 0, 0)),
        out_shape=jax.ShapeDtypeStruct((B, N_CONT, D), jnp.float32),
    )(xc, ln_gamma.reshape(1, N_CONT), ln_beta.reshape(1, N_CONT),
      cont_w, cont_b)


def kernel(X, table, cat_bias, ln_gamma, ln_beta, cont_w, cont_b):
    idx_flat = X[:, :N_CAT].reshape(-1)
    xc = X[:, N_CAT:].astype(jnp.float32)
    x_cat = _cat_call(idx_flat, table, cat_bias).reshape(B, N_CAT, D)
    x_cont = _cont_call(xc, ln_gamma, ln_beta, cont_w, cont_b)
    return x_cat, x_cont


# speedup vs baseline: 1.2042x; 1.2042x over previous
import jax
import jax.numpy as jnp
from jax import lax
from jax.experimental import pallas as pl
from jax.experimental.pallas import tpu as pltpu
from jax.experimental.pallas import tpu_sc as plsc

B = 16384
N_CAT = 26
N_CONT = 13
D = 64

NC = 2
NS = 16
NW = NC * NS
BPW = B // NW


def _cat_sc_kernel(idxT_hbm, table_hbm, bias_hbm, out_hbm,
                   i0, i1, i2, r0, r1, r2, bias_v,
                   g0, g1, g2, w0, w1, w2):
    wid = lax.axis_index("s") * NC + lax.axis_index("c")
    b0 = wid * BPW
    ibufs = (i0, i1, i2)
    rbufs = (r0, r1, r2)
    gsems = (g0, g1, g2)
    wsems = (w0, w1, w2)

    pltpu.sync_copy(bias_hbm, bias_v)

    def fire(j):
        k = j % 3
        pltpu.sync_copy(idxT_hbm.at[j, pl.ds(b0, BPW)], ibufs[k])
        pltpu.make_async_copy(table_hbm.at[ibufs[k]], rbufs[k],
                              gsems[k]).start()

    def wait_gather(j):
        k = j % 3
        pltpu.make_async_copy(table_hbm.at[ibufs[k]], rbufs[k],
                              gsems[k]).wait()

    def write(j):
        k = j % 3
        pltpu.make_async_copy(rbufs[k], out_hbm.at[pl.ds(b0, BPW), j],
                              wsems[k]).start()

    def wait_write(j):
        k = j % 3
        pltpu.make_async_copy(rbufs[k], out_hbm.at[pl.ds(b0, BPW), j],
                              wsems[k]).wait()

    fire(0)
    fire(1)
    for j in range(N_CAT):
        k = j % 3
        if j + 2 < N_CAT:
            if j - 1 >= 0:
                wait_write(j - 1)
            fire(j + 2)
        wait_gather(j)
        rows = rbufs[k]
        bv = [bias_v[j, pl.ds(v * 16, 16)] for v in range(D // 16)]

        def body(r8, carry, rows=rows, bv=bv):
            for u in range(8):
                r = r8 * 8 + u
                for v in range(D // 16):
                    sl = pl.ds(v * 16, 16)
                    rows[r, sl] = rows[r, sl] + bv[v]
            return carry

        lax.fori_loop(0, BPW // 8, body, 0)
        write(j)
    wait_write(N_CAT - 2)
    wait_write(N_CAT - 1)


def _cat_call(idxT, table, cat_bias):
    mesh = plsc.VectorSubcoreMesh(core_axis_name="c", subcore_axis_name="s",
                                  num_cores=NC, num_subcores=NS)
    return pl.kernel(
        _cat_sc_kernel,
        out_type=jax.ShapeDtypeStruct((B, N_CAT, D), jnp.float32),
        mesh=mesh,
        scratch_types=[
            pltpu.VMEM((BPW,), jnp.int32),
            pltpu.VMEM((BPW,), jnp.int32),
            pltpu.VMEM((BPW,), jnp.int32),
            pltpu.VMEM((BPW, D), jnp.float32),
            pltpu.VMEM((BPW, D), jnp.float32),
            pltpu.VMEM((BPW, D), jnp.float32),
            pltpu.VMEM((N_CAT, D), jnp.float32),
            pltpu.SemaphoreType.DMA,
            pltpu.SemaphoreType.DMA,
            pltpu.SemaphoreType.DMA,
            pltpu.SemaphoreType.DMA,
            pltpu.SemaphoreType.DMA,
            pltpu.SemaphoreType.DMA,
        ],
        compiler_params=pltpu.CompilerParams(use_tc_tiling_on_sc=False),
    )(idxT, table, cat_bias)


BLK = 2048


def _cont_tc_kernel(xc_ref, g_ref, b_ref, w_ref, cb_ref, o_ref):
    xc = xc_ref[...]
    mu = jnp.mean(xc, axis=1, keepdims=True)
    var = jnp.mean((xc - mu) ** 2, axis=1, keepdims=True)
    xcn = (xc - mu) * lax.rsqrt(var + 1e-5)
    xcn = xcn * g_ref[...] + b_ref[...]
    for j in range(N_CONT):
        o_ref[:, j, :] = (xcn[:, j:j + 1] * w_ref[j:j + 1, :]
                          + cb_ref[j:j + 1, :])


def _cont_call(xc, ln_gamma, ln_beta, cont_w, cont_b):
    grid = (B // BLK,)
    return pl.pallas_call(
        _cont_tc_kernel,
        grid=grid,
        in_specs=[
            pl.BlockSpec((BLK, N_CONT), lambda i: (i, 0)),
            pl.BlockSpec((1, N_CONT), lambda i: (0, 0)),
            pl.BlockSpec((1, N_CONT), lambda i: (0, 0)),
            pl.BlockSpec((N_CONT, D), lambda i: (0, 0)),
            pl.BlockSpec((N_CONT, D), lambda i: (0, 0)),
        ],
        out_specs=pl.BlockSpec((BLK, N_CONT, D), lambda i: (i, 0, 0)),
        out_shape=jax.ShapeDtypeStruct((B, N_CONT, D), jnp.float32),
    )(xc, ln_gamma.reshape(1, N_CONT), ln_beta.reshape(1, N_CONT),
      cont_w, cont_b)


def kernel(X, table, cat_bias, ln_gamma, ln_beta, cont_w, cont_b):
    idxT = X[:, :N_CAT].T
    xc = X[:, N_CAT:].astype(jnp.float32)
    x_cat = _cat_call(idxT, table, cat_bias)
    x_cont = _cont_call(xc, ln_gamma, ln_beta, cont_w, cont_b)
    return x_cat, x_cont
